# baseline (speedup 1.0000x reference)
import functools

import jax
import jax.numpy as jnp
from jax import lax
from jax.experimental import pallas as pl
from jax.experimental.pallas import tpu as pltpu
from jax.experimental.pallas import tpu_sc as plsc

_TAU = 2048


def _build_sc_weights_kernel(B, E, H, L, NC):
    tail = _TAU * H
    out_e = E + tail
    C = E // 4
    period = H * L
    nper = tail // period

    mesh = plsc.VectorSubcoreMesh(core_axis_name="c", subcore_axis_name="s")

    @functools.partial(
        pl.kernel,
        mesh=mesh,
        out_type=jax.ShapeDtypeStruct((B, 1, out_e), jnp.float32),
        scratch_types=[
            pltpu.VMEM((C,), jnp.float32),
            pltpu.VMEM((tail,), jnp.float32),
            pltpu.SemaphoreType.DMA,
            pltpu.SemaphoreType.DMA,
        ],
    )
    def sc_k(w_hbm, wout_hbm, buf, tl, s0, sem_o):
        c = lax.axis_index("c")
        s = lax.axis_index("s")
        w = s * NC + c
        b = lax.div(w, 4)
        part = lax.rem(w, 4)
        do_tail = part == 0

        pltpu.async_copy(
            w_hbm.at[b, 0, pl.ds(part * C, C)], buf, s0
        )

        @pl.when(do_tail)
        def _zero_tail():
            zero = jnp.zeros((L,), jnp.float32)

            def zbody(ci, carry):
                o = ci * period
                for h in range(H):
                    tl[pl.ds(o + h * L, L)] = zero
                return carry

            lax.fori_loop(0, nper, zbody, 0)
            pltpu.async_copy(tl, wout_hbm.at[b, 0, pl.ds(E, tail)], sem_o)

        pltpu.make_async_copy(w_hbm.at[0, 0, pl.ds(0, C)], buf, s0).wait()
        pltpu.async_copy(buf, wout_hbm.at[b, 0, pl.ds(part * C, C)], sem_o)
        pltpu.make_async_copy(
            buf, wout_hbm.at[0, 0, pl.ds(0, C)], sem_o
        ).wait()

        @pl.when(do_tail)
        def _tail_drain():
            pltpu.make_async_copy(
                tl, wout_hbm.at[0, 0, pl.ds(E, tail)], sem_o
            ).wait()

    return sc_k


def _build_tc_edges_kernel(B, E, H):
    tail = _TAU * H
    out_e = E + tail

    def body(par_ref, e_ref, o_ref):
        o_ref[:, :, pl.ds(0, E)] = e_ref[...]
        b = pl.program_id(0)
        base = par_ref[b]
        j = lax.broadcasted_iota(jnp.int32, (1, 2, tail), 2)
        i = lax.broadcasted_iota(jnp.int32, (1, 2, tail), 1)
        t = j // H
        r = j - t * H
        hop = jnp.full((1, 2, tail), par_ref[B + H - 1], jnp.int32)
        for hh in range(H - 2, -1, -1):
            hop = jnp.where(r == hh, par_ref[B + hh], hop)
        o_ref[:, :, pl.ds(E, tail)] = base + t - i * hop

    return pl.pallas_call(
        body,
        grid=(B,),
        in_specs=[
            pl.BlockSpec(memory_space=pltpu.SMEM),
            pl.BlockSpec((1, 2, E), lambda b: (b, 0, 0)),
        ],
        out_specs=pl.BlockSpec((1, 2, out_e), lambda b: (b, 0, 0)),
        out_shape=jax.ShapeDtypeStruct((B, 2, out_e), jnp.int32),
    )


def kernel(nodes, edges, weights, T, taus, hops):
    del nodes
    B, _, E = edges.shape
    H = hops.shape[0]
    edtype = edges.dtype

    info = plsc.get_sparse_core_info()
    NC, L = info.num_cores, info.num_lanes

    base = T.astype(jnp.int32) + taus.astype(jnp.int32) - _TAU
    params = jnp.concatenate([base, hops.astype(jnp.int32)])

    sc_k = _build_sc_weights_kernel(B, E, H, L, NC)
    weights_out = sc_k(weights)
    edges_out = _build_tc_edges_kernel(B, E, H)(params, edges.astype(jnp.int32))
    return edges_out.astype(edtype), weights_out
